# Initial kernel scaffold; baseline (speedup 1.0000x reference)
import jax
import jax.numpy as jnp
from jax.experimental import pallas as pl


def kernel(x, edge_index, edge_type, W1, W1_root, b1, W2, W2_root, b2):
    raise NotImplementedError("write your pallas kernel here")


# speedup vs baseline: 7.4664x; 7.4664x over previous
import functools

import jax
import jax.numpy as jnp
from jax import lax
from jax.experimental import pallas as pl
from jax.experimental.pallas import tpu as pltpu
from jax.experimental.pallas import tpu_sc as plsc

N = 10000
E = 320000
D = 128
R = 8
NC = 2
NS = 16
NW = NC * NS
NPW = 320
NP = NW * NPW
NSEG = R + 1
HROWS = NSEG * NP
ECAP = E + 128
PR = NPW * R
TRASH = PR
ACCW = (NPW + 1) * D
CHUNK = 2000
GPC = CHUNK // 16

BN = 2048
NT = NP // BN
_F32 = jnp.float32
_I32 = jnp.int32


def _wid():
    return lax.axis_index("s") * NC + lax.axis_index("c")


_MESH = plsc.VectorSubcoreMesh(core_axis_name="c", subcore_axis_name="s")
_SC_PARAMS = pltpu.CompilerParams(needs_layout_passes=False)


@functools.partial(
    pl.kernel,
    mesh=_MESH,
    compiler_params=_SC_PARAMS,
    out_type=(
        jax.ShapeDtypeStruct((NW * PR,), _F32),
        jax.ShapeDtypeStruct((NW * ECAP,), _I32),
        jax.ShapeDtypeStruct((NW * ECAP,), _I32),
        jax.ShapeDtypeStruct((NW * 16,), _I32),
    ),
    scratch_types=[
        pltpu.VMEM((CHUNK,), _I32),
        pltpu.VMEM((CHUNK,), _I32),
        pltpu.VMEM((CHUNK,), _I32),
        pltpu.VMEM((16 * PR,), _F32),
        pltpu.VMEM((PR + 16,), _F32),
        pltpu.VMEM((288,), _I32),
        pltpu.VMEM((288,), _I32),
        pltpu.VMEM((16,), _I32),
    ],
)
def _sc_prep(src_h, dst_h, typ_h, rcp_h, gl_h, pl_h, mb_h,
             csrc, cdst, ctyp, cnts, rbuf, sg, sp, mbuf):
    wid = _wid()
    lo = wid * NPW
    lane = lax.iota(_I32, 16)
    zero16 = jnp.zeros((16,), _F32)
    one16 = jnp.ones((16,), _F32)

    def zbody(i, _):
        cnts[pl.ds(i * 16, 16)] = zero16
        return 0
    lax.fori_loop(0, PR, zbody, 0)

    def chunk_body(c, carry):
        pltpu.sync_copy(src_h.at[pl.ds(c * CHUNK, CHUNK)], csrc)
        pltpu.sync_copy(dst_h.at[pl.ds(c * CHUNK, CHUNK)], cdst)
        pltpu.sync_copy(typ_h.at[pl.ds(c * CHUNK, CHUNK)], ctyp)

        def grp(g, carry2):
            wptr, mcur = carry2
            s = csrc[pl.ds(g * 16, 16)]
            d = cdst[pl.ds(g * 16, 16)]
            t = ctyp[pl.ds(g * 16, 16)]
            dl = d - lo
            m = (dl >= 0) & (dl < NPW)
            gi = t * NP + s
            pi = dl * R + t
            pic = jnp.clip(pi, 0, PR - 1)
            plsc.addupdate_scatter(cnts, [lane * PR + pic], one16, mask=m)
            plsc.store_compressed(sg.at[pl.ds(wptr, 16)], gi, mask=m)
            plsc.store_compressed(sp.at[pl.ds(wptr, 16)], pi, mask=m)
            wptr = wptr + jnp.sum(m.astype(_I32))

            def do_flush(args):
                w, mc = args
                pltpu.sync_copy(sg.at[pl.ds(0, 128)],
                                gl_h.at[pl.ds(wid * ECAP + mc * 128, 128)])
                pltpu.sync_copy(sp.at[pl.ds(0, 128)],
                                pl_h.at[pl.ds(wid * ECAP + mc * 128, 128)])
                sg[pl.ds(0, 16)] = sg[pl.ds(128, 16)]
                sp[pl.ds(0, 16)] = sp[pl.ds(128, 16)]
                return (w - 128, mc + 1)

            return lax.cond(wptr >= 128, do_flush, lambda a: a, (wptr, mcur))

        return lax.fori_loop(0, GPC, grp, carry)

    wptr, mcur = lax.fori_loop(0, E // CHUNK, chunk_body,
                               (jnp.int32(0), jnp.int32(0)))

    zi16 = jnp.zeros((16,), _I32)
    tr16 = jnp.full((16,), TRASH, _I32)
    for g in range(8):
        sg[pl.ds(wptr + g * 16, 16)] = zi16
        sp[pl.ds(wptr + g * 16, 16)] = tr16
    pltpu.sync_copy(sg.at[pl.ds(0, 128)],
                    gl_h.at[pl.ds(wid * ECAP + mcur * 128, 128)])
    pltpu.sync_copy(sp.at[pl.ds(0, 128)],
                    pl_h.at[pl.ds(wid * ECAP + mcur * 128, 128)])
    mbuf[pl.ds(0, 16)] = jnp.full((16,), mcur + 1, _I32)
    pltpu.sync_copy(mbuf, mb_h.at[pl.ds(wid * 16, 16)])

    def red(i, _):
        a = zero16
        for l in range(16):
            a = a + cnts[pl.ds(l * PR + i * 16, 16)]
        rbuf[pl.ds(i * 16, 16)] = 1.0 / jnp.maximum(a, 1.0)
        return 0
    lax.fori_loop(0, PR // 16, red, 0)
    pltpu.sync_copy(rbuf.at[pl.ds(0, PR)], rcp_h.at[pl.ds(wid * PR, PR)])


@functools.partial(
    pl.kernel,
    mesh=_MESH,
    compiler_params=_SC_PARAMS,
    out_type=jax.ShapeDtypeStruct((NW * NPW * D,), _F32),
    scratch_types=[
        pltpu.VMEM((PR + 16,), _F32),
        pltpu.VMEM((ACCW,), _F32),
        pltpu.VMEM((128,), _I32),
        pltpu.VMEM((128,), _I32),
        pltpu.VMEM((128, D), _F32),
        pltpu.VMEM((16,), _I32),
        pltpu.SemaphoreType.DMA,
    ],
)
def _sc_msg(h_h, gl_h, pl_h, mb_h, rcp_h, msg_h,
            rbuf, acc, gb, pb, rows, mb, sem):
    wid = _wid()
    base = wid * ECAP
    zero16 = jnp.zeros((16,), _F32)

    pltpu.sync_copy(mb_h.at[pl.ds(wid * 16, 16)], mb)
    pltpu.sync_copy(rcp_h.at[pl.ds(wid * PR, PR)], rbuf.at[pl.ds(0, PR)])
    rbuf[pl.ds(PR, 16)] = zero16
    nblk = mb[...][0]

    def zbody(i, _):
        acc[pl.ds(i * 16, 16)] = zero16
        return 0
    lax.fori_loop(0, ACCW // 16, zbody, 0)

    def blk(k, _):
        pltpu.sync_copy(gl_h.at[pl.ds(base + k * 128, 128)], gb)
        pltpu.sync_copy(pl_h.at[pl.ds(base + k * 128, 128)], pb)
        pltpu.async_copy(h_h.at[gb], rows, sem).wait()

        def grp(g, _):
            pvec = pb[pl.ds(g * 16, 16)]
            rcv = plsc.load_gather(rbuf, [pvec])
            rbv = lax.shift_left(lax.shift_right_logical(pvec, 3), 7)
            for jj in range(16):
                j = g * 16 + jj
                rowb = rbv[jj]
                rv = jnp.full((16,), rcv[jj])
                for p in range(8):
                    v = rows[j, pl.ds(p * 16, 16)]
                    plsc.addupdate(acc.at[pl.ds(rowb + p * 16, 16)], v * rv)
            return 0
        lax.fori_loop(0, 8, grp, 0)
        return 0
    lax.fori_loop(0, nblk, blk, 0)

    pltpu.sync_copy(acc.at[pl.ds(0, NPW * D)],
                    msg_h.at[pl.ds(wid * NPW * D, NPW * D)])


def _tc_mm(x_pad, wcat):
    def body(x_ref, w_ref, o_ref):
        o_ref[...] = jnp.dot(x_ref[...], w_ref[0],
                             preferred_element_type=_F32)
    return pl.pallas_call(
        body,
        grid=(NT, NSEG),
        in_specs=[
            pl.BlockSpec((BN, D), lambda t, r: (t, 0)),
            pl.BlockSpec((1, D, D), lambda t, r: (r, 0, 0)),
        ],
        out_specs=pl.BlockSpec((BN, D), lambda t, r: (r * NT + t, 0)),
        out_shape=jax.ShapeDtypeStruct((HROWS, D), _F32),
    )(x_pad, wcat)


def _tc_mm_fused(h_prev, b_prev, msg_prev, wcat):
    def body(h8_ref, b_ref, m_ref, w_ref, o_ref):
        o1 = jnp.maximum(h8_ref[...] + b_ref[...] + m_ref[...], 0.0)
        o_ref[...] = jnp.dot(o1, w_ref[0], preferred_element_type=_F32)
    return pl.pallas_call(
        body,
        grid=(NT, NSEG),
        in_specs=[
            pl.BlockSpec((BN, D), lambda t, r: (R * NT + t, 0)),
            pl.BlockSpec((1, D), lambda t, r: (0, 0)),
            pl.BlockSpec((BN, D), lambda t, r: (t, 0)),
            pl.BlockSpec((1, D, D), lambda t, r: (r, 0, 0)),
        ],
        out_specs=pl.BlockSpec((BN, D), lambda t, r: (r * NT + t, 0)),
        out_shape=jax.ShapeDtypeStruct((HROWS, D), _F32),
    )(h_prev, b_prev, msg_prev, wcat)


def _tc_combine(h_prev, b_prev, msg_prev):
    def body(h8_ref, b_ref, m_ref, o_ref):
        o_ref[...] = h8_ref[...] + b_ref[...] + m_ref[...]
    return pl.pallas_call(
        body,
        grid=(NT,),
        in_specs=[
            pl.BlockSpec((BN, D), lambda t: (R * NT + t, 0)),
            pl.BlockSpec((1, D), lambda t: (0, 0)),
            pl.BlockSpec((BN, D), lambda t: (t, 0)),
        ],
        out_specs=pl.BlockSpec((BN, D), lambda t: (t, 0)),
        out_shape=jax.ShapeDtypeStruct((NP, D), _F32),
    )(h_prev, b_prev, msg_prev)


def kernel(x, edge_index, edge_type, W1, W1_root, b1, W2, W2_root, b2):
    src = edge_index[0]
    dst = edge_index[1]
    x_pad = jnp.zeros((NP, D), _F32).at[:N].set(x)
    wcat1 = jnp.concatenate([W1, W1_root[None]], axis=0)
    wcat2 = jnp.concatenate([W2, W2_root[None]], axis=0)
    b1r = b1.reshape(1, D)
    b2r = b2.reshape(1, D)

    rcp, gl, plst, mb = _sc_prep(src, dst, edge_type)

    h1 = _tc_mm(x_pad, wcat1)
    msg1 = _sc_msg(h1, gl, plst, mb, rcp).reshape(NP, D)
    h2 = _tc_mm_fused(h1, b1r, msg1, wcat2)
    msg2 = _sc_msg(h2, gl, plst, mb, rcp).reshape(NP, D)
    out = _tc_combine(h2, b2r, msg2)
    return out[:N]
